# baseline (speedup 1.0000x reference)
import functools

import jax
import jax.numpy as jnp
from jax import lax
from jax.experimental import pallas as pl
from jax.experimental.pallas import tpu as pltpu
from jax.experimental.pallas import tpu_sc as plsc

D = 768
LANES = 16
NVREG = D // LANES
EPS = 1e-12

NUM_CORES = 2
NUM_SUBCORES = 16
NW = NUM_CORES * NUM_SUBCORES

R = 16
NBUF = 2
RI = 2
DIAG_NO_DMA = False


def _shuffle(v, idx):
    dnums = lax.GatherDimensionNumbers(
        offset_dims=(), collapsed_slice_dims=(0,), start_index_map=(0,))
    return lax.gather(v, idx[:, None], dnums, slice_sizes=(1,),
                      mode=lax.GatherScatterMode.PROMISE_IN_BOUNDS)


def _lane_sum(v):
    lane = lax.iota(jnp.int32, LANES)
    for shift in (8, 4, 2, 1):
        v = v + _shuffle(v, lane ^ shift)
    return v


def _vrsqrt(x):
    i = lax.bitcast_convert_type(x, jnp.int32)
    i = jnp.int32(0x5F3759DF) - (i >> 1)
    y = lax.bitcast_convert_type(i, jnp.float32)
    for _ in range(3):
        y = y * (1.5 - 0.5 * x * y * y)
    return y


def _make_sc_kernel(n_tokens):
    tpw = n_tokens // NW
    n_chunks = tpw // R
    mesh = plsc.VectorSubcoreMesh(core_axis_name="c", subcore_axis_name="s")

    @functools.partial(
        pl.kernel,
        out_type=jax.ShapeDtypeStruct((n_tokens, D), jnp.float32),
        mesh=mesh,
        scratch_types=[
            pltpu.VMEM((tpw,), jnp.int32),
            pltpu.VMEM((tpw,), jnp.int32),
            pltpu.VMEM((D,), jnp.float32),
            [pltpu.VMEM((R, D), jnp.float32)] * NBUF,
            [pltpu.VMEM((R, D), jnp.float32)] * NBUF,
            [pltpu.VMEM((R, D), jnp.float32)] * NBUF,
            pltpu.VMEM((R, D), jnp.float32),
            [pltpu.SemaphoreType.DMA] * NBUF,
            [pltpu.SemaphoreType.DMA] * NBUF,
            [pltpu.SemaphoreType.DMA] * NBUF,
        ],
    )
    def sc_embed(ids_hbm, pos_hbm, wt_hbm, pt_hbm, nw_hbm, out_hbm,
                 idw, idp, nwv, wbufs, pbufs, obufs, sbuf,
                 sems_w, sems_p, sems_o):
        wid = lax.axis_index("s") * NUM_CORES + lax.axis_index("c")
        base = wid * tpw
        pltpu.sync_copy(ids_hbm.at[pl.ds(base, tpw)], idw)
        pltpu.sync_copy(pos_hbm.at[pl.ds(base, tpw)], idp)
        pltpu.sync_copy(nw_hbm, nwv)

        def gather_copies(c, b):
            cw = pltpu.make_async_copy(
                wt_hbm.at[idw.at[pl.ds(c * R, R)]], wbufs[b], sems_w[b])
            cp = pltpu.make_async_copy(
                pt_hbm.at[idp.at[pl.ds(c * R, R)]], pbufs[b], sems_p[b])
            return cw, cp

        def scatter_copy(c, b):
            return pltpu.make_async_copy(
                obufs[b], out_hbm.at[pl.ds(base + c * R, R)], sems_o[b])

        if not DIAG_NO_DMA:
            for b in range(NBUF):
                cw, cp = gather_copies(b, b)
                cw.start()
                cp.start()

        def compute_chunk(wb, pb, ob):
            lane = lax.iota(jnp.int32, LANES)
            zero = jnp.zeros((LANES,), jnp.float32)

            def p1_body(j, accs):
                sl = pl.ds(j * LANES, LANES)
                new = []
                for r in range(R):
                    s = wb[r, sl] + pb[r, sl]
                    sbuf[r, sl] = s
                    new.append(accs[r] + s * s)
                return tuple(new)

            accs = lax.fori_loop(0, NVREG, p1_body, (zero,) * R)

            sums = zero
            for r in range(R):
                sums = jnp.where(lane == r, _lane_sum(accs[r]), sums)
            scalevec = _vrsqrt(sums * (1.0 / D) + EPS)

            splats = tuple(
                _shuffle(scalevec, jnp.full((LANES,), r, jnp.int32))
                for r in range(R))

            def p2_body(j, sv):
                sl = pl.ds(j * LANES, LANES)
                w = nwv[sl]
                for r in range(R):
                    ob[r, sl] = sbuf[r, sl] * sv[r] * w
                return sv

            lax.fori_loop(0, NVREG, p2_body, splats)

        def process_chunk(c, b, first, last):
            if DIAG_NO_DMA:
                compute_chunk(wbufs[b], pbufs[b], obufs[b])
                return
            cw, cp = gather_copies(c, b)
            cw.wait()
            cp.wait()

            @pl.when(jnp.logical_not(first))
            def _():
                scatter_copy(c - NBUF, b).wait()

            compute_chunk(wbufs[b], pbufs[b], obufs[b])

            if last is not True:
                @pl.when(jnp.logical_not(last))
                def _():
                    nw_, np_ = gather_copies(c + NBUF, b)
                    nw_.start()
                    np_.start()

            scatter_copy(c, b).start()

        def iter_body(i, carry):
            for b in range(NBUF):
                c = i * NBUF + b
                process_chunk(c, b, c < NBUF, c + NBUF >= n_chunks)
            return carry

        n_loop = n_chunks // NBUF
        lax.fori_loop(0, n_loop, iter_body, 0)
        for c in range(n_loop * NBUF, n_chunks):
            process_chunk(c, c % NBUF, c < NBUF, c + NBUF >= n_chunks)

        if not DIAG_NO_DMA:
            for c in range(n_chunks - NBUF, n_chunks):
                scatter_copy(c, c % NBUF).wait()

    return sc_embed


def kernel(input_ids, position_idcs, word_table, pos_table, norm_weight):
    batch, seq = input_ids.shape
    n_tokens = batch * seq
    ids = input_ids.reshape(n_tokens).astype(jnp.int32)
    pos = position_idcs.reshape(n_tokens).astype(jnp.int32)
    sc = _make_sc_kernel(n_tokens)
    out = sc(ids, pos, word_table, pos_table, norm_weight)
    return out.reshape(batch, seq, D)


# speedup vs baseline: 1.0821x; 1.0821x over previous
import functools

import jax
import jax.numpy as jnp
from jax import lax
from jax.experimental import pallas as pl
from jax.experimental.pallas import tpu as pltpu
from jax.experimental.pallas import tpu_sc as plsc

D = 768
LANES = 16
NVREG = D // LANES
EPS = 1e-12

NUM_CORES = 2
NUM_SUBCORES = 16
NW = NUM_CORES * NUM_SUBCORES

R = 16
NBUF = 2


def _shuffle(v, idx):
    dnums = lax.GatherDimensionNumbers(
        offset_dims=(), collapsed_slice_dims=(0,), start_index_map=(0,))
    return lax.gather(v, idx[:, None], dnums, slice_sizes=(1,),
                      mode=lax.GatherScatterMode.PROMISE_IN_BOUNDS)


def _lane_sum(v):
    lane = lax.iota(jnp.int32, LANES)
    for shift in (8, 4, 2, 1):
        v = v + _shuffle(v, lane ^ shift)
    return v


def _vrsqrt(x):
    i = lax.bitcast_convert_type(x, jnp.int32)
    i = jnp.int32(0x5F3759DF) - (i >> 1)
    y = lax.bitcast_convert_type(i, jnp.float32)
    for _ in range(3):
        y = y * (1.5 - 0.5 * x * y * y)
    return y


def _make_sc_kernel(n_tokens):
    tpw = n_tokens // NW
    n_chunks = tpw // R
    mesh = plsc.VectorSubcoreMesh(core_axis_name="c", subcore_axis_name="s")

    @functools.partial(
        pl.kernel,
        out_type=jax.ShapeDtypeStruct((n_tokens, D), jnp.float32),
        mesh=mesh,
        scratch_types=[
            pltpu.VMEM((tpw,), jnp.int32),
            pltpu.VMEM((tpw,), jnp.int32),
            pltpu.VMEM((D,), jnp.float32),
            [pltpu.VMEM((R, D), jnp.float32)] * NBUF,
            [pltpu.VMEM((R, D), jnp.float32)] * NBUF,
            [pltpu.VMEM((R, D), jnp.float32)] * NBUF,
            [pltpu.VMEM((R, D), jnp.float32)] * NBUF,
            [pltpu.SemaphoreType.DMA] * NBUF,
            [pltpu.SemaphoreType.DMA] * NBUF,
            [pltpu.SemaphoreType.DMA] * NBUF,
        ],
    )
    def sc_embed(ids_hbm, pos_hbm, wt_hbm, pt_hbm, nw_hbm, out_hbm,
                 idw, idp, nwv, wbufs, pbufs, obufs, sbufs,
                 sems_w, sems_p, sems_o):
        wid = lax.axis_index("s") * NUM_CORES + lax.axis_index("c")
        base = wid * tpw
        pltpu.sync_copy(ids_hbm.at[pl.ds(base, tpw)], idw)
        pltpu.sync_copy(pos_hbm.at[pl.ds(base, tpw)], idp)
        pltpu.sync_copy(nw_hbm, nwv)

        def gather_copies(c, b):
            cw = pltpu.make_async_copy(
                wt_hbm.at[idw.at[pl.ds(c * R, R)]], wbufs[b], sems_w[b])
            cp = pltpu.make_async_copy(
                pt_hbm.at[idp.at[pl.ds(c * R, R)]], pbufs[b], sems_p[b])
            return cw, cp

        def scatter_copy(c, b):
            return pltpu.make_async_copy(
                obufs[b], out_hbm.at[pl.ds(base + c * R, R)], sems_o[b])

        for b in range(NBUF):
            cw, cp = gather_copies(b, b)
            cw.start()
            cp.start()

        lane = lax.iota(jnp.int32, LANES)
        zero = jnp.zeros((LANES,), jnp.float32)

        def p1_loop(wb, pb, sb):
            def body(j, accs):
                sl = pl.ds(j * LANES, LANES)
                new = []
                for r in range(R):
                    s = wb[r, sl] + pb[r, sl]
                    sb[r, sl] = s
                    new.append(accs[r] + s * s)
                return tuple(new)

            return lax.fori_loop(0, NVREG, body, (zero,) * R)

        def fused_loop(wb, pb, sb_cur, sb_prev, ob_prev, sp):
            def body(j, carry):
                accs = carry[:R]
                sl = pl.ds(j * LANES, LANES)
                w = nwv[sl]
                new = []
                for r in range(R):
                    s = wb[r, sl] + pb[r, sl]
                    sb_cur[r, sl] = s
                    new.append(accs[r] + s * s)
                for r in range(R):
                    ob_prev[r, sl] = sb_prev[r, sl] * carry[R + r] * w
                return tuple(new) + carry[R:]

            out = lax.fori_loop(0, NVREG, body, (zero,) * R + sp)
            return out[:R]

        def p2_loop(sb, ob, sp):
            def body(j, sv):
                sl = pl.ds(j * LANES, LANES)
                w = nwv[sl]
                for r in range(R):
                    ob[r, sl] = sb[r, sl] * sv[r] * w
                return sv

            lax.fori_loop(0, NVREG, body, sp)

        def make_splats(accs):
            sums = zero
            for r in range(R):
                sums = jnp.where(lane == r, _lane_sum(accs[r]), sums)
            scalevec = _vrsqrt(sums * (1.0 / D) + EPS)
            return tuple(
                _shuffle(scalevec, jnp.full((LANES,), r, jnp.int32))
                for r in range(R))

        def step(c, b, sp):
            cw, cp = gather_copies(c, b)
            cw.wait()
            cp.wait()

            @pl.when(c >= 3)
            def _():
                scatter_copy(c - 3, 1 - b).wait()

            accs = fused_loop(wbufs[b], pbufs[b], sbufs[b], sbufs[1 - b],
                              obufs[1 - b], sp)
            new_sp = make_splats(accs)
            scatter_copy(c - 1, 1 - b).start()
            if not (isinstance(c, int) and c + 2 >= n_chunks):
                @pl.when(c + 2 < n_chunks)
                def _():
                    nw_, np_ = gather_copies(c + 2, b)
                    nw_.start()
                    np_.start()
            return new_sp

        cw, cp = gather_copies(0, 0)
        cw.wait()
        cp.wait()
        sp = make_splats(p1_loop(wbufs[0], pbufs[0], sbufs[0]))
        nw_, np_ = gather_copies(2, 0)
        nw_.start()
        np_.start()

        def iter_body(i, sp):
            sp = step(2 * i + 1, 1, sp)
            sp = step(2 * i + 2, 0, sp)
            return sp

        sp = lax.fori_loop(0, (n_chunks - 2) // 2, iter_body, sp)
        sp = step(n_chunks - 1, (n_chunks - 1) % 2, sp)

        lastb = (n_chunks - 1) % 2
        scatter_copy(n_chunks - 3, lastb).wait()
        p2_loop(sbufs[lastb], obufs[lastb], sp)
        scatter_copy(n_chunks - 1, lastb).start()
        scatter_copy(n_chunks - 2, 1 - lastb).wait()
        scatter_copy(n_chunks - 1, lastb).wait()

    return sc_embed


def kernel(input_ids, position_idcs, word_table, pos_table, norm_weight):
    batch, seq = input_ids.shape
    n_tokens = batch * seq
    ids = input_ids.reshape(n_tokens).astype(jnp.int32)
    pos = position_idcs.reshape(n_tokens).astype(jnp.int32)
    sc = _make_sc_kernel(n_tokens)
    out = sc(ids, pos, word_table, pos_table, norm_weight)
    return out.reshape(batch, seq, D)


# speedup vs baseline: 1.1488x; 1.0617x over previous
import functools

import jax
import jax.numpy as jnp
from jax import lax
from jax.experimental import pallas as pl
from jax.experimental.pallas import tpu as pltpu
from jax.experimental.pallas import tpu_sc as plsc

D = 768
LANES = 16
NVREG = D // LANES
EPS = 1e-12

NUM_CORES = 2
NUM_SUBCORES = 16
NW = NUM_CORES * NUM_SUBCORES

R = 16
NBUF = 2


def _shuffle(v, idx):
    dnums = lax.GatherDimensionNumbers(
        offset_dims=(), collapsed_slice_dims=(0,), start_index_map=(0,))
    return lax.gather(v, idx[:, None], dnums, slice_sizes=(1,),
                      mode=lax.GatherScatterMode.PROMISE_IN_BOUNDS)


def _lane_sum(v):
    lane = lax.iota(jnp.int32, LANES)
    for shift in (8, 4, 2, 1):
        v = v + _shuffle(v, lane ^ shift)
    return v


def _vrsqrt(x):
    i = lax.bitcast_convert_type(x, jnp.int32)
    i = jnp.int32(0x5F3759DF) - (i >> 1)
    y = lax.bitcast_convert_type(i, jnp.float32)
    for _ in range(2):
        y = y * (1.5 - 0.5 * x * y * y)
    return y


def _make_sc_kernel(n_tokens):
    tpw = n_tokens // NW
    n_chunks = tpw // R
    mesh = plsc.VectorSubcoreMesh(core_axis_name="c", subcore_axis_name="s")

    @functools.partial(
        pl.kernel,
        out_type=jax.ShapeDtypeStruct((n_tokens, D), jnp.float32),
        mesh=mesh,
        scratch_types=[
            pltpu.VMEM((tpw,), jnp.int32),
            pltpu.VMEM((tpw,), jnp.int32),
            pltpu.VMEM((D,), jnp.float32),
            [pltpu.VMEM((R, D), jnp.float32)] * NBUF,
            [pltpu.VMEM((R, D), jnp.float32)] * NBUF,
            [pltpu.VMEM((R, D), jnp.float32)] * NBUF,
            [pltpu.VMEM((R, D), jnp.float32)] * NBUF,
            pltpu.VMEM((LANES,), jnp.float32),
            [pltpu.SemaphoreType.DMA] * NBUF,
            [pltpu.SemaphoreType.DMA] * NBUF,
            [pltpu.SemaphoreType.DMA] * NBUF,
        ],
    )
    def sc_embed(ids_hbm, pos_hbm, wt_hbm, pt_hbm, nw_hbm, out_hbm,
                 idw, idp, nwv, wbufs, pbufs, obufs, sbufs, scalebuf,
                 sems_w, sems_p, sems_o):
        wid = lax.axis_index("s") * NUM_CORES + lax.axis_index("c")
        base = wid * tpw
        pltpu.sync_copy(ids_hbm.at[pl.ds(base, tpw)], idw)
        pltpu.sync_copy(pos_hbm.at[pl.ds(base, tpw)], idp)
        pltpu.sync_copy(nw_hbm, nwv)

        def gather_copies(c, b):
            cw = pltpu.make_async_copy(
                wt_hbm.at[idw.at[pl.ds(c * R, R)]], wbufs[b], sems_w[b])
            cp = pltpu.make_async_copy(
                pt_hbm.at[idp.at[pl.ds(c * R, R)]], pbufs[b], sems_p[b])
            return cw, cp

        def scatter_copy(c, b):
            return pltpu.make_async_copy(
                obufs[b], out_hbm.at[pl.ds(base + c * R, R)], sems_o[b])

        for b in range(NBUF):
            cw, cp = gather_copies(b, b)
            cw.start()
            cp.start()

        lane = lax.iota(jnp.int32, LANES)
        zero = jnp.zeros((LANES,), jnp.float32)

        def p1_loop(wb, pb, sb):
            def body(j, accs):
                sl = pl.ds(j * LANES, LANES)
                new = []
                for r in range(R):
                    s = wb[r, sl] + pb[r, sl]
                    sb[r, sl] = s
                    new.append(accs[r] + s * s)
                return tuple(new)

            return lax.fori_loop(0, NVREG, body, (zero,) * R)

        def _splat(sv, r):
            return jnp.full((LANES,), sv[r], jnp.float32)

        def fused_loop(wb, pb, sb_cur, sb_prev, ob_prev):
            def body(j, carry):
                accs, sv = carry[:R], carry[R]
                sl = pl.ds(j * LANES, LANES)
                w = nwv[sl]
                new = []
                for r in range(R):
                    s = wb[r, sl] + pb[r, sl]
                    sb_cur[r, sl] = s
                    new.append(accs[r] + s * s)
                    ob_prev[r, sl] = sb_prev[r, sl] * _splat(sv, r) * w
                return tuple(new) + (sv,)

            out = lax.fori_loop(0, NVREG, body,
                                (zero,) * R + (scalebuf[:],))
            return out[:R]

        def p2_loop(sb, ob):
            def body(j, sv):
                sl = pl.ds(j * LANES, LANES)
                w = nwv[sl]
                for r in range(R):
                    ob[r, sl] = sb[r, sl] * _splat(sv, r) * w
                return sv

            lax.fori_loop(0, NVREG, body, scalebuf[:])

        def store_scales(accs):
            sums = zero
            for r in range(R):
                sums = jnp.where(lane == r, _lane_sum(accs[r]), sums)
            scalebuf[:] = _vrsqrt(sums * (1.0 / D) + EPS)

        def step(c, b):
            cw, cp = gather_copies(c, b)
            cw.wait()
            cp.wait()

            @pl.when(c >= 3)
            def _():
                scatter_copy(c - 3, 1 - b).wait()

            accs = fused_loop(wbufs[b], pbufs[b], sbufs[b], sbufs[1 - b],
                              obufs[1 - b])
            store_scales(accs)
            scatter_copy(c - 1, 1 - b).start()
            if not (isinstance(c, int) and c + 2 >= n_chunks):
                @pl.when(c + 2 < n_chunks)
                def _():
                    nw_, np_ = gather_copies(c + 2, b)
                    nw_.start()
                    np_.start()

        cw, cp = gather_copies(0, 0)
        cw.wait()
        cp.wait()
        store_scales(p1_loop(wbufs[0], pbufs[0], sbufs[0]))
        nw_, np_ = gather_copies(2, 0)
        nw_.start()
        np_.start()

        def iter_body(i, carry):
            step(2 * i + 1, 1)
            step(2 * i + 2, 0)
            return carry

        lax.fori_loop(0, (n_chunks - 2) // 2, iter_body, 0)
        step(n_chunks - 1, (n_chunks - 1) % 2)

        lastb = (n_chunks - 1) % 2
        scatter_copy(n_chunks - 3, lastb).wait()
        p2_loop(sbufs[lastb], obufs[lastb])
        scatter_copy(n_chunks - 1, lastb).start()
        scatter_copy(n_chunks - 2, 1 - lastb).wait()
        scatter_copy(n_chunks - 1, lastb).wait()

    return sc_embed


def kernel(input_ids, position_idcs, word_table, pos_table, norm_weight):
    batch, seq = input_ids.shape
    n_tokens = batch * seq
    ids = input_ids.reshape(n_tokens).astype(jnp.int32)
    pos = position_idcs.reshape(n_tokens).astype(jnp.int32)
    sc = _make_sc_kernel(n_tokens)
    out = sc(ids, pos, word_table, pos_table, norm_weight)
    return out.reshape(batch, seq, D)


# speedup vs baseline: 1.1550x; 1.0054x over previous
import functools

import jax
import jax.numpy as jnp
from jax import lax
from jax.experimental import pallas as pl
from jax.experimental.pallas import tpu as pltpu
from jax.experimental.pallas import tpu_sc as plsc

D = 768
LANES = 16
NVREG = D // LANES
EPS = 1e-12

NUM_CORES = 2
NUM_SUBCORES = 16
NW = NUM_CORES * NUM_SUBCORES

R = 16
NBUF = 2


def _shuffle(v, idx):
    dnums = lax.GatherDimensionNumbers(
        offset_dims=(), collapsed_slice_dims=(0,), start_index_map=(0,))
    return lax.gather(v, idx[:, None], dnums, slice_sizes=(1,),
                      mode=lax.GatherScatterMode.PROMISE_IN_BOUNDS)


def _lane_sum(v):
    lane = lax.iota(jnp.int32, LANES)
    for shift in (8, 4, 2, 1):
        v = v + _shuffle(v, lane ^ shift)
    return v


def _vrsqrt(x):
    i = lax.bitcast_convert_type(x, jnp.int32)
    i = jnp.int32(0x5F3759DF) - (i >> 1)
    y = lax.bitcast_convert_type(i, jnp.float32)
    for _ in range(2):
        y = y * (1.5 - 0.5 * x * y * y)
    return y


def _make_sc_kernel(n_tokens):
    tpw = n_tokens // NW
    n_chunks = tpw // R
    mesh = plsc.VectorSubcoreMesh(core_axis_name="c", subcore_axis_name="s")

    @functools.partial(
        pl.kernel,
        out_type=jax.ShapeDtypeStruct((n_tokens, D), jnp.float32),
        mesh=mesh,
        scratch_types=[
            pltpu.VMEM((tpw,), jnp.int32),
            pltpu.VMEM((tpw,), jnp.int32),
            pltpu.VMEM((D,), jnp.float32),
            [pltpu.VMEM((R, D), jnp.float32)] * NBUF,
            [pltpu.VMEM((R, D), jnp.float32)] * NBUF,
            [pltpu.VMEM((R, D), jnp.float32)] * NBUF,
            [pltpu.VMEM((R, D), jnp.float32)] * NBUF,
            pltpu.VMEM((LANES,), jnp.float32),
            [pltpu.SemaphoreType.DMA] * NBUF,
            [pltpu.SemaphoreType.DMA] * NBUF,
            [pltpu.SemaphoreType.DMA] * NBUF,
        ],
    )
    def sc_embed(ids_hbm, pos_hbm, wt_hbm, pt_hbm, nw_hbm, out_hbm,
                 idw, idp, nwv, wbufs, pbufs, obufs, sbufs, scalebuf,
                 sems_w, sems_p, sems_o):
        wid = lax.axis_index("s") * NUM_CORES + lax.axis_index("c")
        base = wid * tpw
        pltpu.sync_copy(ids_hbm.at[pl.ds(base, tpw)], idw)
        pltpu.sync_copy(pos_hbm.at[pl.ds(base, tpw)], idp)
        pltpu.sync_copy(nw_hbm, nwv)

        def gather_copies(c, b):
            cw = pltpu.make_async_copy(
                wt_hbm.at[idw.at[pl.ds(c * R, R)]], wbufs[b], sems_w[b])
            cp = pltpu.make_async_copy(
                pt_hbm.at[idp.at[pl.ds(c * R, R)]], pbufs[b], sems_p[b])
            return cw, cp

        def scatter_copy(c, b):
            return pltpu.make_async_copy(
                obufs[b], out_hbm.at[pl.ds(base + c * R, R)], sems_o[b])

        for b in range(NBUF):
            cw, cp = gather_copies(b, b)
            cw.start()
            cp.start()

        lane = lax.iota(jnp.int32, LANES)
        zero = jnp.zeros((LANES,), jnp.float32)

        def p1_loop(wb, pb, sb):
            def body(j, accs):
                sl = pl.ds(j * LANES, LANES)
                new = []
                for r in range(R):
                    s = wb[r, sl] + pb[r, sl]
                    sb[r, sl] = s
                    new.append(accs[r] + s * s)
                return tuple(new)

            return lax.fori_loop(0, NVREG, body, (zero,) * R)

        def _splat(sv, r):
            return jnp.full((LANES,), sv[r], jnp.float32)

        def fused_loop(wb, pb, sb_cur, sb_prev, ob_prev):
            def body(j, carry):
                accs, sv = carry[:R], carry[R]
                sl = pl.ds(j * LANES, LANES)
                w = nwv[sl]
                new = []
                for r in range(R):
                    s = wb[r, sl] + pb[r, sl]
                    sb_cur[r, sl] = s
                    new.append(accs[r] + s * s)
                    ob_prev[r, sl] = sb_prev[r, sl] * _splat(sv, r) * w
                return tuple(new) + (sv,)

            out = lax.fori_loop(0, NVREG, body,
                                (zero,) * R + (scalebuf[:],))
            return out[:R]

        def p2_loop(sb, ob):
            def body(j, sv):
                sl = pl.ds(j * LANES, LANES)
                w = nwv[sl]
                for r in range(R):
                    ob[r, sl] = sb[r, sl] * _splat(sv, r) * w
                return sv

            lax.fori_loop(0, NVREG, body, scalebuf[:])

        def store_scales(accs):
            vs = list(accs)
            k = 1
            while len(vs) > 1:
                nxt = []
                pick = (lane & k) == 0
                for i in range(0, len(vs), 2):
                    su = vs[i] + _shuffle(vs[i], lane ^ k)
                    sv = vs[i + 1] + _shuffle(vs[i + 1], lane ^ k)
                    nxt.append(jnp.where(pick, su, sv))
                vs = nxt
                k *= 2
            scalebuf[:] = _vrsqrt(vs[0] * (1.0 / D) + EPS)

        def step(c, b):
            cw, cp = gather_copies(c, b)
            cw.wait()
            cp.wait()

            @pl.when(c >= 3)
            def _():
                scatter_copy(c - 3, 1 - b).wait()

            accs = fused_loop(wbufs[b], pbufs[b], sbufs[b], sbufs[1 - b],
                              obufs[1 - b])
            store_scales(accs)
            scatter_copy(c - 1, 1 - b).start()
            if not (isinstance(c, int) and c + 2 >= n_chunks):
                @pl.when(c + 2 < n_chunks)
                def _():
                    nw_, np_ = gather_copies(c + 2, b)
                    nw_.start()
                    np_.start()

        cw, cp = gather_copies(0, 0)
        cw.wait()
        cp.wait()
        store_scales(p1_loop(wbufs[0], pbufs[0], sbufs[0]))
        nw_, np_ = gather_copies(2, 0)
        nw_.start()
        np_.start()

        def iter_body(i, carry):
            step(2 * i + 1, 1)
            step(2 * i + 2, 0)
            return carry

        lax.fori_loop(0, (n_chunks - 2) // 2, iter_body, 0)
        step(n_chunks - 1, (n_chunks - 1) % 2)

        lastb = (n_chunks - 1) % 2
        scatter_copy(n_chunks - 3, lastb).wait()
        p2_loop(sbufs[lastb], obufs[lastb])
        scatter_copy(n_chunks - 1, lastb).start()
        scatter_copy(n_chunks - 2, 1 - lastb).wait()
        scatter_copy(n_chunks - 1, lastb).wait()

    return sc_embed


def kernel(input_ids, position_idcs, word_table, pos_table, norm_weight):
    batch, seq = input_ids.shape
    n_tokens = batch * seq
    ids = input_ids.reshape(n_tokens).astype(jnp.int32)
    pos = position_idcs.reshape(n_tokens).astype(jnp.int32)
    sc = _make_sc_kernel(n_tokens)
    out = sc(ids, pos, word_table, pos_table, norm_weight)
    return out.reshape(batch, seq, D)


# speedup vs baseline: 1.1559x; 1.0008x over previous
import functools

import jax
import jax.numpy as jnp
from jax import lax
from jax.experimental import pallas as pl
from jax.experimental.pallas import tpu as pltpu
from jax.experimental.pallas import tpu_sc as plsc

D = 768
LANES = 16
NVREG = D // LANES
EPS = 1e-12

NUM_CORES = 2
NUM_SUBCORES = 16
NW = NUM_CORES * NUM_SUBCORES

R = 16
NBUF = 2


def _shuffle(v, idx):
    dnums = lax.GatherDimensionNumbers(
        offset_dims=(), collapsed_slice_dims=(0,), start_index_map=(0,))
    return lax.gather(v, idx[:, None], dnums, slice_sizes=(1,),
                      mode=lax.GatherScatterMode.PROMISE_IN_BOUNDS)


def _vrsqrt(x):
    i = lax.bitcast_convert_type(x, jnp.int32)
    i = jnp.int32(0x5F3759DF) - (i >> 1)
    y = lax.bitcast_convert_type(i, jnp.float32)
    for _ in range(2):
        y = y * (1.5 - 0.5 * x * y * y)
    return y


def _make_sc_kernel(n_tokens):
    tpw = n_tokens // NW
    n_chunks = tpw // R
    mesh = plsc.VectorSubcoreMesh(core_axis_name="c", subcore_axis_name="s")

    @functools.partial(
        pl.kernel,
        out_type=jax.ShapeDtypeStruct((n_tokens, D), jnp.float32),
        mesh=mesh,
        scratch_types=[
            pltpu.VMEM((tpw,), jnp.int32),
            pltpu.VMEM((tpw,), jnp.int32),
            pltpu.VMEM((D,), jnp.float32),
            [pltpu.VMEM((R, D), jnp.float32)] * NBUF,
            [pltpu.VMEM((R, D), jnp.float32)] * NBUF,
            [pltpu.VMEM((R, D), jnp.float32)] * NBUF,
            [pltpu.VMEM((R, D), jnp.float32)] * NBUF,
            pltpu.VMEM((LANES,), jnp.float32),
            [pltpu.SemaphoreType.DMA] * NBUF,
            [pltpu.SemaphoreType.DMA] * NBUF,
            [pltpu.SemaphoreType.DMA] * NBUF,
        ],
    )
    def sc_embed(ids_hbm, pos_hbm, wt_hbm, pt_hbm, nw_hbm, out_hbm,
                 idw, idp, nwv, wbufs, pbufs, obufs, sbufs, scalebuf,
                 sems_w, sems_p, sems_o):
        wid = lax.axis_index("s") * NUM_CORES + lax.axis_index("c")
        base = wid * tpw
        pltpu.sync_copy(ids_hbm.at[pl.ds(base, tpw)], idw)
        pltpu.sync_copy(pos_hbm.at[pl.ds(base, tpw)], idp)
        pltpu.sync_copy(nw_hbm, nwv)

        def gather_copies(c, b):
            cw = pltpu.make_async_copy(
                wt_hbm.at[idw.at[pl.ds(c * R, R)]], wbufs[b], sems_w[b])
            cp = pltpu.make_async_copy(
                pt_hbm.at[idp.at[pl.ds(c * R, R)]], pbufs[b], sems_p[b])
            return cw, cp

        def scatter_copy(c, b):
            return pltpu.make_async_copy(
                obufs[b], out_hbm.at[pl.ds(base + c * R, R)], sems_o[b])

        for b in range(NBUF):
            cw, cp = gather_copies(b, b)
            cw.start()
            cp.start()

        lane = lax.iota(jnp.int32, LANES)
        zero = jnp.zeros((LANES,), jnp.float32)

        def p1_loop(wb, pb, sb):
            def body(j, accs):
                sl = pl.ds(j * LANES, LANES)
                new = []
                for r in range(R):
                    s = wb[r, sl] + pb[r, sl]
                    sb[r, sl] = s
                    new.append(accs[r] + s * s)
                return tuple(new)

            return lax.fori_loop(0, NVREG, body, (zero,) * R)

        def _splat(sv, r):
            return jnp.full((LANES,), sv[r], jnp.float32)

        def fused_loop(wb, pb, sb_cur, sb_prev, ob_prev):
            def body(j, carry):
                accs, sv = carry[:R], carry[R]
                sl = pl.ds(j * LANES, LANES)
                w = nwv[sl]
                new = []
                for r in range(R):
                    s = wb[r, sl] + pb[r, sl]
                    sb_cur[r, sl] = s
                    new.append(accs[r] + s * s)
                    ob_prev[r, sl] = sb_prev[r, sl] * _splat(sv, r) * w
                return tuple(new) + (sv,)

            out = lax.fori_loop(0, NVREG, body,
                                (zero,) * R + (scalebuf[:],))
            return out[:R]

        def p2_loop(sb, ob):
            def body(j, sv):
                sl = pl.ds(j * LANES, LANES)
                w = nwv[sl]
                for r in range(R):
                    ob[r, sl] = sb[r, sl] * _splat(sv, r) * w
                return sv

            lax.fori_loop(0, NVREG, body, scalebuf[:])

        def store_scales(accs):
            vs = list(accs)
            k = 1
            while len(vs) > 1:
                nxt = []
                pick = (lane & k) == 0
                for i in range(0, len(vs), 2):
                    su = vs[i] + _shuffle(vs[i], lane ^ k)
                    sv = vs[i + 1] + _shuffle(vs[i + 1], lane ^ k)
                    nxt.append(jnp.where(pick, su, sv))
                vs = nxt
                k *= 2
            scalebuf[:] = _vrsqrt(vs[0] * (1.0 / D) + EPS)

        def step(c, b):
            cw, cp = gather_copies(c, b)
            cw.wait()
            cp.wait()

            @pl.when(c >= 3)
            def _():
                scatter_copy(c - 3, 1 - b).wait()

            accs = fused_loop(wbufs[b], pbufs[b], sbufs[b], sbufs[1 - b],
                              obufs[1 - b])
            store_scales(accs)
            scatter_copy(c - 1, 1 - b).start()
            if not (isinstance(c, int) and c + 2 >= n_chunks):
                @pl.when(c + 2 < n_chunks)
                def _():
                    nw_, np_ = gather_copies(c + 2, b)
                    nw_.start()
                    np_.start()

        cw, cp = gather_copies(0, 0)
        cw.wait()
        cp.wait()
        store_scales(p1_loop(wbufs[0], pbufs[0], sbufs[0]))
        nw_, np_ = gather_copies(2, 0)
        nw_.start()
        np_.start()

        def iter_body(i, carry):
            step(2 * i + 1, 1)
            step(2 * i + 2, 0)
            return carry

        lax.fori_loop(0, (n_chunks - 2) // 2, iter_body, 0)
        step(n_chunks - 1, (n_chunks - 1) % 2)

        lastb = (n_chunks - 1) % 2
        scatter_copy(n_chunks - 3, lastb).wait()
        p2_loop(sbufs[lastb], obufs[lastb])
        scatter_copy(n_chunks - 1, lastb).start()
        scatter_copy(n_chunks - 2, 1 - lastb).wait()
        scatter_copy(n_chunks - 1, lastb).wait()

    return sc_embed


def kernel(input_ids, position_idcs, word_table, pos_table, norm_weight):
    batch, seq = input_ids.shape
    n_tokens = batch * seq
    ids = input_ids.reshape(n_tokens).astype(jnp.int32)
    pos = position_idcs.reshape(n_tokens).astype(jnp.int32)
    sc = _make_sc_kernel(n_tokens)
    out = sc(ids, pos, word_table, pos_table, norm_weight)
    return out.reshape(batch, seq, D)
